# baseline (speedup 1.0000x reference)
import functools

import jax
import jax.numpy as jnp
from jax import lax
from jax.experimental import pallas as pl
from jax.experimental.pallas import tpu as pltpu
from jax.experimental.pallas import tpu_sc as plsc

_LANES = 16
_NUM_CORES = 2
_NUM_SUBCORES = 16
_NUM_WORKERS = _NUM_CORES * _NUM_SUBCORES
_CHUNK = 128
_NVAR = 16


@functools.lru_cache(maxsize=None)
def _build_sc_kernel(S: int, num_buckets: int, num_heads: int):
    L = _LANES
    NW = _NUM_WORKERS
    rows_total = num_heads * S
    assert rows_total % NW == 0
    rows_per_worker = rows_total // NW
    assert rows_per_worker % 64 == 0 and S % rows_per_worker == 0
    P = 2 * S
    assert P % _CHUNK == 0
    md = num_buckets // 2
    shift = md - (S - 1)

    mesh = plsc.VectorSubcoreMesh(core_axis_name="c", subcore_axis_name="s")

    @functools.partial(
        pl.kernel,
        mesh=mesh,
        out_type=jax.ShapeDtypeStruct((rows_total, S), jnp.float32),
        compiler_params=pltpu.CompilerParams(use_tc_tiling_on_sc=False),
        scratch_types=[
            pltpu.VMEM((P,), jnp.int32),
            pltpu.VMEM((_NVAR * P,), jnp.float32),
            pltpu.SemaphoreType.DMA,
        ],
    )
    def sc_kernel(bt_hbm, out_hbm, idx_v, pw_v, sem):
        wid = lax.axis_index("s") * _NUM_CORES + lax.axis_index("c")
        row0 = wid * rows_per_worker
        h = row0 // S
        i0 = row0 - h * S

        iota = lax.iota(jnp.int32, L)
        hbase = h * num_buckets

        def variant(r, _):
            def build_idx(slot, _b):
                base_u = slot * L
                c0 = (base_u + shift + r) + iota
                idx_v[pl.ds(base_u, L)] = hbase + jnp.clip(c0, 0, num_buckets - 1)
                return 0

            lax.fori_loop(0, P // L, build_idx, 0, unroll=False)

            def gather(g, _b):
                handles = []
                for b in range(8):
                    off = (g * 8 + b) * _CHUNK
                    src = bt_hbm.at[idx_v.at[pl.ds(off, _CHUNK)]]
                    dst = pw_v.at[pl.ds(r * P + off, _CHUNK)]
                    handles.append(pltpu.async_copy(src, dst, sem))
                for hd in handles:
                    hd.wait()
                return 0

            lax.fori_loop(0, P // _CHUNK // 8, gather, 0, unroll=False)
            return 0

        lax.fori_loop(0, _NVAR, variant, 0, unroll=False)

        def fire_row(i):
            start = (S - 1) - i
            r = jnp.bitwise_and(start, _NVAR - 1)
            off = pl.multiple_of(r * P + (start - r), _NVAR)
            src = pw_v.at[pl.ds(off, S)]
            dst = out_hbm.at[h * S + i]
            return pltpu.async_copy(src, dst, sem)

        DEPTH = 32
        B = 8
        for b in range(DEPTH):
            fire_row(i0 + b)

        def rows(g, _):
            i_base = i0 + DEPTH + g * B
            handles = [fire_row(i_base + b) for b in range(B)]
            for hd in handles:
                hd.wait()
            return 0

        lax.fori_loop(0, (rows_per_worker - DEPTH) // B, rows, 0, unroll=False)
        for b in range(DEPTH):
            pltpu.make_async_copy(
                out_hbm.at[h * S + i0], pw_v.at[pl.ds(0, S)], sem
            ).wait()

    return sc_kernel


def kernel(x, bias_values):
    S = x.shape[1]
    num_buckets, num_heads = bias_values.shape
    sc = _build_sc_kernel(S, num_buckets, num_heads)
    bt = bias_values.astype(jnp.float32).T.reshape(-1)
    out = sc(bt)
    return out.reshape(1, num_heads, S, S)


# speedup vs baseline: 1.4242x; 1.4242x over previous
import functools

import jax
import jax.numpy as jnp
from jax import lax
from jax.experimental import pallas as pl
from jax.experimental.pallas import tpu as pltpu
from jax.experimental.pallas import tpu_sc as plsc

_LANES = 16
_NUM_CORES = 2
_NUM_SUBCORES = 16
_NUM_WORKERS = _NUM_CORES * _NUM_SUBCORES
_CHUNK = 128
_NVAR = 8


@functools.lru_cache(maxsize=None)
def _build_sc_kernel(S: int, num_buckets: int, num_heads: int):
    L = _LANES
    NW = _NUM_WORKERS
    rows_total = num_heads * S
    assert rows_total % NW == 0
    rows_per_worker = rows_total // NW
    assert rows_per_worker % 64 == 0 and S % rows_per_worker == 0
    assert _NVAR * 2 * S % (8 * _CHUNK) == 0
    P = 2 * S
    assert P % _CHUNK == 0
    md = num_buckets // 2
    shift = md - (S - 1)

    mesh = plsc.VectorSubcoreMesh(core_axis_name="c", subcore_axis_name="s")

    @functools.partial(
        pl.kernel,
        mesh=mesh,
        out_type=jax.ShapeDtypeStruct((1, num_heads, S, S), jnp.float32),
        compiler_params=pltpu.CompilerParams(use_tc_tiling_on_sc=False),
        scratch_types=[
            pltpu.VMEM((_NVAR * P,), jnp.int32),
            pltpu.VMEM((_NVAR * P,), jnp.float32),
            pltpu.SemaphoreType.DMA,
        ],
    )
    def sc_kernel(bt_hbm, out_hbm, idx_v, pw_v, sem):
        wid = lax.axis_index("s") * _NUM_CORES + lax.axis_index("c")
        row0 = wid * rows_per_worker
        h = row0 // S
        i0 = row0 - h * S

        iota = lax.iota(jnp.int32, L)
        hbase = h * num_buckets

        def build_idx(slot, _):
            base_u = slot * L
            c0 = (base_u + shift) + iota
            for r in range(_NVAR):
                idx_v[pl.ds(r * P + base_u, L)] = hbase + jnp.clip(
                    c0 + r, 0, num_buckets - 1
                )
            return 0

        lax.fori_loop(0, P // L, build_idx, 0, unroll=False)

        def gather(g, _):
            handles = []
            for b in range(8):
                off = (g * 8 + b) * _CHUNK
                src = bt_hbm.at[idx_v.at[pl.ds(off, _CHUNK)]]
                handles.append(pltpu.async_copy(src, pw_v.at[pl.ds(off, _CHUNK)], sem))
            for hd in handles:
                hd.wait()
            return 0

        lax.fori_loop(0, _NVAR * P // _CHUNK // 8, gather, 0, unroll=False)

        def fire_row(i):
            start = (S - 1) - i
            r = jnp.bitwise_and(start, _NVAR - 1)
            off = pl.multiple_of(r * P + (start - r), _NVAR)
            src = pw_v.at[pl.ds(off, S)]
            dst = out_hbm.at[0, h, i]
            return pltpu.async_copy(src, dst, sem)

        DEPTH = 32
        B = 8
        for b in range(DEPTH):
            fire_row(i0 + b)

        def rows(g, _):
            i_base = i0 + DEPTH + g * B
            handles = [fire_row(i_base + b) for b in range(B)]
            for hd in handles:
                hd.wait()
            return 0

        lax.fori_loop(0, (rows_per_worker - DEPTH) // B, rows, 0, unroll=False)
        for b in range(DEPTH):
            pltpu.make_async_copy(
                out_hbm.at[0, h, i0], pw_v.at[pl.ds(0, S)], sem
            ).wait()

    return sc_kernel


def kernel(x, bias_values):
    S = x.shape[1]
    num_buckets, num_heads = bias_values.shape
    sc = _build_sc_kernel(S, num_buckets, num_heads)
    bt = bias_values.astype(jnp.float32).T.reshape(-1)
    return sc(bt)


# speedup vs baseline: 1.4515x; 1.0192x over previous
import functools

import jax
import jax.numpy as jnp
from jax import lax
from jax.experimental import pallas as pl
from jax.experimental.pallas import tpu as pltpu
from jax.experimental.pallas import tpu_sc as plsc

_LANES = 16
_NUM_CORES = 2
_NUM_SUBCORES = 16
_NUM_WORKERS = _NUM_CORES * _NUM_SUBCORES
_CHUNK = 128
_NVAR = 8


@functools.lru_cache(maxsize=None)
def _build_sc_kernel(S: int, num_buckets: int, num_heads: int):
    L = _LANES
    NW = _NUM_WORKERS
    rows_total = num_heads * S
    assert rows_total % NW == 0
    rows_per_worker = rows_total // NW
    assert rows_per_worker % 64 == 0 and S % rows_per_worker == 0
    assert _NVAR * 2 * S % (8 * _CHUNK) == 0
    P = 2 * S
    assert P % _CHUNK == 0
    md = num_buckets // 2
    shift = md - (S - 1)

    mesh = plsc.VectorSubcoreMesh(core_axis_name="c", subcore_axis_name="s")

    @functools.partial(
        pl.kernel,
        mesh=mesh,
        out_type=jax.ShapeDtypeStruct((1, num_heads, S, S), jnp.float32),
        compiler_params=pltpu.CompilerParams(use_tc_tiling_on_sc=False),
        scratch_types=[
            pltpu.VMEM((_NVAR * P,), jnp.int32),
            pltpu.VMEM((_NVAR * P,), jnp.float32),
            pltpu.SemaphoreType.DMA,
        ],
    )
    def sc_kernel(bt_hbm, out_hbm, idx_v, pw_v, sem):
        wid = lax.axis_index("s") * _NUM_CORES + lax.axis_index("c")
        row0 = wid * rows_per_worker
        h = row0 // S
        i0 = row0 - h * S

        iota = lax.iota(jnp.int32, L)
        hbase = h * num_buckets

        def build_idx(slot, _):
            base_u = slot * L
            c0 = (base_u + shift) + iota
            for r in range(_NVAR):
                idx_v[pl.ds(r * P + base_u, L)] = hbase + jnp.clip(
                    c0 + r, 0, num_buckets - 1
                )
            return 0

        lax.fori_loop(0, P // L, build_idx, 0, unroll=False)

        n_chunks = _NVAR * P // _CHUNK

        def fire_chunk(c):
            off = c * _CHUNK
            src = bt_hbm.at[idx_v.at[pl.ds(off, _CHUNK)]]
            return pltpu.async_copy(src, pw_v.at[pl.ds(off, _CHUNK)], sem)

        GDEPTH = 32
        GB = 8
        for b in range(GDEPTH):
            fire_chunk(b)

        def gather(g, _):
            cb = GDEPTH + g * GB
            handles = [fire_chunk(cb + b) for b in range(GB)]
            for hd in handles:
                hd.wait()
            return 0

        lax.fori_loop(0, (n_chunks - GDEPTH) // GB, gather, 0, unroll=False)
        for b in range(GDEPTH):
            pltpu.make_async_copy(
                bt_hbm.at[pl.ds(0, _CHUNK)], pw_v.at[pl.ds(0, _CHUNK)], sem
            ).wait()

        def fire_row(i):
            start = (S - 1) - i
            r = jnp.bitwise_and(start, _NVAR - 1)
            off = pl.multiple_of(r * P + (start - r), _NVAR)
            src = pw_v.at[pl.ds(off, S)]
            dst = out_hbm.at[0, h, i]
            return pltpu.async_copy(src, dst, sem)

        DEPTH = 32
        B = 8
        for b in range(DEPTH):
            fire_row(i0 + b)

        def rows(g, _):
            i_base = i0 + DEPTH + g * B
            handles = [fire_row(i_base + b) for b in range(B)]
            for hd in handles:
                hd.wait()
            return 0

        lax.fori_loop(0, (rows_per_worker - DEPTH) // B, rows, 0, unroll=False)
        for b in range(DEPTH):
            pltpu.make_async_copy(
                out_hbm.at[0, h, i0], pw_v.at[pl.ds(0, S)], sem
            ).wait()

    return sc_kernel


def kernel(x, bias_values):
    S = x.shape[1]
    num_buckets, num_heads = bias_values.shape
    sc = _build_sc_kernel(S, num_buckets, num_heads)
    bt = bias_values.astype(jnp.float32).T.reshape(-1)
    return sc(bt)


# speedup vs baseline: 2.7565x; 1.8991x over previous
import functools

import jax
import jax.numpy as jnp
from jax import lax
from jax.experimental import pallas as pl
from jax.experimental.pallas import tpu as pltpu
from jax.experimental.pallas import tpu_sc as plsc

_LANES = 16
_NUM_CORES = 2
_NUM_SUBCORES = 16
_NUM_WORKERS = _NUM_CORES * _NUM_SUBCORES
_CHUNK = 128
_NVAR = 8


@functools.lru_cache(maxsize=None)
def _build_sc_kernel(S: int, num_buckets: int, num_heads: int):
    L = _LANES
    NW = _NUM_WORKERS
    rows_total = num_heads * S
    assert rows_total % NW == 0
    rows_per_worker = rows_total // NW
    assert rows_per_worker % 64 == 0 and S % rows_per_worker == 0
    assert S % 128 == 0
    P = 2 * S
    assert (_NVAR * P) % (8 * _CHUNK) == 0 and P % _CHUNK == 0
    md = num_buckets // 2
    shift = md - (S - 1)

    mesh = plsc.VectorSubcoreMesh(core_axis_name="c", subcore_axis_name="s")

    @functools.partial(
        pl.kernel,
        mesh=mesh,
        out_type=jax.ShapeDtypeStruct((rows_total // 8, S // 128, 8, 128), jnp.float32),
        compiler_params=pltpu.CompilerParams(use_tc_tiling_on_sc=False),
        scratch_types=[
            pltpu.VMEM((_NVAR * P,), jnp.int32),
            pltpu.VMEM((_NVAR, P), jnp.float32),
            pltpu.SemaphoreType.DMA,
        ],
    )
    def sc_kernel(bt_hbm, out_hbm, idx_v, pw_v, sem):
        wid = lax.axis_index("s") * _NUM_CORES + lax.axis_index("c")
        row0 = wid * rows_per_worker
        h = row0 // S
        i0 = row0 - h * S

        iota = lax.iota(jnp.int32, L)
        hbase = h * num_buckets

        def build_idx(slot, _):
            base_u = slot * L
            c0 = (base_u + shift) + iota
            for k in range(_NVAR):
                idx_v[pl.ds(k * P + base_u, L)] = hbase + jnp.clip(
                    c0 + (_NVAR - 1 - k), 0, num_buckets - 1
                )
            return 0

        lax.fori_loop(0, P // L, build_idx, 0, unroll=False)

        n_chunks = _NVAR * P // _CHUNK
        per_row = P // _CHUNK

        def fire_chunk(c):
            k = c // per_row
            off = (c - k * per_row) * _CHUNK
            src = bt_hbm.at[idx_v.at[pl.ds(k * P + off, _CHUNK)]]
            return pltpu.async_copy(src, pw_v.at[k, pl.ds(off, _CHUNK)], sem)

        GDEPTH = 32
        GB = 8
        for b in range(GDEPTH):
            fire_chunk(b)

        def gather(g, _):
            cb = GDEPTH + g * GB
            handles = [fire_chunk(cb + b) for b in range(GB)]
            for hd in handles:
                hd.wait()
            return 0

        lax.fori_loop(0, (n_chunks - GDEPTH) // GB, gather, 0, unroll=False)
        for b in range(GDEPTH):
            pltpu.make_async_copy(
                bt_hbm.at[pl.ds(0, _CHUNK)], pw_v.at[0, pl.ds(0, _CHUNK)], sem
            ).wait()

        n_cblk = S // 128
        rb0 = (h * S + i0) // 8

        def fire_tile(t):
            q = t // n_cblk
            c = t - q * n_cblk
            ib = i0 + 8 * q
            w0 = pl.multiple_of((S - 8) - ib + 128 * c, 8)
            src = pw_v.at[:, pl.ds(w0, 128)]
            dst = out_hbm.at[rb0 + q, c]
            return pltpu.async_copy(src, dst, sem)

        n_tiles = (rows_per_worker // 8) * n_cblk
        DEPTH = 32
        B = 8
        for b in range(DEPTH):
            fire_tile(b)

        def tiles(g, _):
            tb = DEPTH + g * B
            handles = [fire_tile(tb + b) for b in range(B)]
            for hd in handles:
                hd.wait()
            return 0

        lax.fori_loop(0, (n_tiles - DEPTH) // B, tiles, 0, unroll=False)
        for b in range(DEPTH):
            pltpu.make_async_copy(
                out_hbm.at[rb0, 0], pw_v.at[:, pl.ds(0, 128)], sem
            ).wait()

    return sc_kernel


def kernel(x, bias_values):
    S = x.shape[1]
    num_buckets, num_heads = bias_values.shape
    sc = _build_sc_kernel(S, num_buckets, num_heads)
    bt = bias_values.astype(jnp.float32).T.reshape(-1)
    out4 = sc(bt)
    out = (
        out4.reshape(num_heads, S // 8, S // 128, 8, 128)
        .transpose(0, 1, 3, 2, 4)
        .reshape(1, num_heads, S, S)
    )
    return out
